# baseline (speedup 1.0000x reference)
import jax
import jax.numpy as jnp
from jax import lax
from jax.experimental import pallas as pl
from jax.experimental.pallas import tpu as pltpu
from jax.experimental.pallas import tpu_sc as plsc

N_NODES = 10000
N_PAD = 10240
E_EDGES = 320000
D_FEAT = 128

NUM_CORES = 2
NUM_SUBCORES = 16
NUM_TILES = NUM_CORES * NUM_SUBCORES
CHUNK = 128
E_PAD = 327680
N_CHUNKS_TOT = E_PAD // CHUNK
ROWS_PER_TILE = N_PAD // NUM_SUBCORES

DEG_CHUNKS = N_CHUNKS_TOT // NUM_TILES

M0 = 92
M1 = N_CHUNKS_TOT // NUM_SUBCORES - M0
GRP = 46
NBUF = 2


def _deg_body(row2d_hbm, out_hbm, deg_sh, zbuf_v, rowidx_v, ones_v, dsem):
    c = lax.axis_index("c")
    s = lax.axis_index("s")
    wid = c * NUM_SUBCORES + s
    r0 = s * ROWS_PER_TILE

    zeros16 = jnp.zeros((16,), dtype=jnp.float32)

    def zbody(i, _):
        zbuf_v[pl.ds(i * 16, 16)] = zeros16
        return 0
    lax.fori_loop(0, ROWS_PER_TILE // 16, zbody, 0)
    for j in range(CHUNK // 16):
        ones_v[pl.ds(j * 16, 16)] = jnp.full((16,), 1.0, dtype=jnp.float32)
    pltpu.sync_copy(zbuf_v, deg_sh.at[pl.ds(r0, ROWS_PER_TILE)])

    pltpu.sync_copy(row2d_hbm.at[pl.ds(wid * DEG_CHUNKS, DEG_CHUNKS)],
                    rowidx_v)
    plsc.subcore_barrier()

    K = 8

    def body(g, _):
        descs = [
            pltpu.async_copy(ones_v, deg_sh.at[rowidx_v.at[g * K + k]],
                             dsem, add=True)
            for k in range(K)
        ]
        for d in descs:
            d.wait()
        return 0
    lax.fori_loop(0, DEG_CHUNKS // K, body, 0)

    plsc.subcore_barrier()
    pltpu.sync_copy(deg_sh.at[pl.ds(r0, ROWS_PER_TILE)],
                    out_hbm.at[c, pl.ds(r0, ROWS_PER_TILE)])


def _sc_degree(row2d):
    mesh = plsc.VectorSubcoreMesh(core_axis_name="c", subcore_axis_name="s")
    return pl.kernel(
        _deg_body,
        out_type=jax.ShapeDtypeStruct((NUM_CORES, N_PAD), jnp.float32),
        mesh=mesh,
        scratch_types=[
            pltpu.VMEM_SHARED((N_PAD,), jnp.float32),
            pltpu.VMEM((ROWS_PER_TILE,), jnp.float32),
            pltpu.VMEM((DEG_CHUNKS, CHUNK), jnp.int32),
            pltpu.VMEM((CHUNK,), jnp.float32),
            pltpu.SemaphoreType.DMA,
        ],
        compiler_params=pltpu.CompilerParams(needs_layout_passes=False,
                                             use_tc_tiling_on_sc=False),
    )(row2d)


BLK = 512


def _dis_col(degp_blk):
    deg = jnp.sum(jnp.transpose(degp_blk), axis=1, keepdims=True)
    return jnp.where(deg > 0.0, lax.rsqrt(deg), 0.0)


def _prep_body(degp_ref, x_ref, xs_ref):
    xs_ref[...] = _dis_col(degp_ref[...]) * x_ref[...]


def _tc_prep(degp, x_pad):
    grid = (N_PAD // BLK,)
    return pl.pallas_call(
        _prep_body,
        grid=grid,
        in_specs=[
            pl.BlockSpec((NUM_CORES, BLK), lambda i: (0, i)),
            pl.BlockSpec((BLK, D_FEAT), lambda i: (i, 0)),
        ],
        out_specs=pl.BlockSpec((BLK, D_FEAT), lambda i: (i, 0)),
        out_shape=jax.ShapeDtypeStruct((N_PAD, D_FEAT), jnp.float32),
    )(degp, x_pad)


def _scat_body(xs_hbm, row2d_hbm, col2d_hbm, zeros_hbm, out_hbm,
               acc_sh, colidx_v, rowidx_v, gbufs, gsems, ssems):
    c = lax.axis_index("c")
    s = lax.axis_index("s")
    r0 = s * ROWS_PER_TILE

    pltpu.sync_copy(zeros_hbm.at[pl.ds(r0, ROWS_PER_TILE)],
                    acc_sh.at[pl.ds(r0, ROWS_PER_TILE)])
    plsc.subcore_barrier()

    def gather(j, b):
        pltpu.async_copy(xs_hbm.at[colidx_v.at[j]], gbufs.at[b], gsems.at[b])

    def gather_wait(j, b):
        pltpu.make_async_copy(xs_hbm.at[colidx_v.at[j]], gbufs.at[b],
                              gsems.at[b]).wait()

    def scat(j, b):
        pltpu.async_copy(gbufs.at[b], acc_sh.at[rowidx_v.at[j]],
                         ssems.at[b], add=True)

    def scat_wait(j, b):
        pltpu.make_async_copy(gbufs.at[b], acc_sh.at[rowidx_v.at[j]],
                              ssems.at[b]).wait()

    def run_groups(base_chunk, m, grp):
        ngrp = m // grp

        def grp_body(g, _):
            g0 = base_chunk + g * grp
            pltpu.sync_copy(col2d_hbm.at[pl.ds(g0, grp)],
                            colidx_v.at[pl.ds(0, grp)])
            pltpu.sync_copy(row2d_hbm.at[pl.ds(g0, grp)],
                            rowidx_v.at[pl.ds(0, grp)])

            gather(0, 0)

            def turn(j, _):
                b = lax.rem(j, NBUF)
                nb = lax.rem(j + 1, NBUF)

                @pl.when(j + 1 < grp)
                def _():
                    @pl.when(j >= 1)
                    def _():
                        scat_wait(j - 1, nb)
                    gather(j + 1, nb)

                gather_wait(j, b)
                scat(j, b)
                return 0
            lax.fori_loop(0, grp, turn, 0)
            scat_wait(grp - 1, (grp - 1) % NBUF)
            return 0
        lax.fori_loop(0, ngrp, grp_body, 0)

    @pl.when(c == 0)
    def _():
        run_groups(s * M0, M0, M0 // 2)

    @pl.when(c == 1)
    def _():
        run_groups(NUM_SUBCORES * M0 + s * M1, M1, M1 // 2)

    plsc.subcore_barrier()
    pltpu.sync_copy(acc_sh.at[pl.ds(r0, ROWS_PER_TILE)],
                    out_hbm.at[c, pl.ds(r0, ROWS_PER_TILE)])


def _sc_scatter(xs, row2d, col2d, zeros):
    mesh = plsc.VectorSubcoreMesh(core_axis_name="c", subcore_axis_name="s")
    return pl.kernel(
        _scat_body,
        out_type=jax.ShapeDtypeStruct((NUM_CORES, N_PAD, D_FEAT), jnp.float32),
        mesh=mesh,
        scratch_types=[
            pltpu.VMEM_SHARED((N_PAD, D_FEAT), jnp.float32),
            pltpu.VMEM((GRP, CHUNK), jnp.int32),
            pltpu.VMEM((GRP, CHUNK), jnp.int32),
            pltpu.VMEM((NBUF, CHUNK, D_FEAT), jnp.float32),
            pltpu.SemaphoreType.DMA((NBUF,)),
            pltpu.SemaphoreType.DMA((NBUF,)),
        ],
        compiler_params=pltpu.CompilerParams(needs_layout_passes=False,
                                             use_tc_tiling_on_sc=False),
    )(xs, row2d, col2d, zeros)


def _final_body(degp_ref, x_ref, a0_ref, a1_ref, w0_ref, w1_ref, b_ref,
                out_ref):
    acc = a0_ref[0] + a1_ref[0]
    tx1 = -_dis_col(degp_ref[...]) * acc
    o = (jnp.dot(x_ref[...], w0_ref[...],
                 preferred_element_type=jnp.float32)
         + jnp.dot(tx1, w1_ref[...],
                   preferred_element_type=jnp.float32)
         + b_ref[...])
    out_ref[...] = jnp.where(o > 0.0, o, jnp.exp(jnp.minimum(o, 0.0)) - 1.0)


def _tc_final(degp, x_pad, acc, W0, W1, b2d):
    grid = (N_PAD // BLK,)
    return pl.pallas_call(
        _final_body,
        grid=grid,
        in_specs=[
            pl.BlockSpec((NUM_CORES, BLK), lambda i: (0, i)),
            pl.BlockSpec((BLK, D_FEAT), lambda i: (i, 0)),
            pl.BlockSpec((1, BLK, D_FEAT), lambda i: (0, i, 0)),
            pl.BlockSpec((1, BLK, D_FEAT), lambda i: (1, i, 0)),
            pl.BlockSpec((D_FEAT, D_FEAT), lambda i: (0, 0)),
            pl.BlockSpec((D_FEAT, D_FEAT), lambda i: (0, 0)),
            pl.BlockSpec((1, D_FEAT), lambda i: (0, 0)),
        ],
        out_specs=pl.BlockSpec((BLK, D_FEAT), lambda i: (i, 0)),
        out_shape=jax.ShapeDtypeStruct((N_PAD, D_FEAT), jnp.float32),
    )(degp, x_pad, acc, acc, W0, W1, b2d)


@jax.jit
def kernel(x, edge_index, W0, W1, b):
    row = edge_index[0]
    col = edge_index[1]
    pad_e = E_PAD - E_EDGES
    row2d = jnp.concatenate(
        [row, jnp.full((pad_e,), N_NODES, dtype=jnp.int32)]
    ).reshape(N_CHUNKS_TOT, CHUNK)
    col2d = jnp.concatenate(
        [col, jnp.full((pad_e,), N_NODES, dtype=jnp.int32)]
    ).reshape(N_CHUNKS_TOT, CHUNK)
    x_pad = jnp.pad(x, ((0, N_PAD - N_NODES), (0, 0)))
    zeros = jnp.zeros((N_PAD, D_FEAT), dtype=jnp.float32)

    degp = _sc_degree(row2d)
    xs = _tc_prep(degp, x_pad)
    acc = _sc_scatter(xs, row2d, col2d, zeros)
    out_pad = _tc_final(degp, x_pad, acc, W0, W1, b.reshape(1, D_FEAT))
    return out_pad[:N_NODES]


# speedup vs baseline: 1.3517x; 1.3517x over previous
import jax
import jax.numpy as jnp
from jax import lax
from jax.experimental import pallas as pl
from jax.experimental.pallas import tpu as pltpu
from jax.experimental.pallas import tpu_sc as plsc

N_NODES = 10000
N_PAD = 10240
E_EDGES = 320000
D_FEAT = 128
D_HALF = 64

NUM_CORES = 2
NUM_SUBCORES = 16
NUM_TILES = NUM_CORES * NUM_SUBCORES
CHUNK = 128
E_PAD = 327680
ROWS_PER_TILE = N_PAD // NUM_SUBCORES

DEG_CHUNKS = E_PAD // (NUM_TILES * CHUNK)

SC_CHUNKS = E_PAD // (NUM_SUBCORES * CHUNK)
NBUF = 5
LOOKAHEAD = 2


def _deg_body(row2d_hbm, out_hbm, deg_sh, zbuf_v, rowidx_v, ones_v, dsem):
    c = lax.axis_index("c")
    s = lax.axis_index("s")
    wid = c * NUM_SUBCORES + s
    r0 = s * ROWS_PER_TILE

    zeros16 = jnp.zeros((16,), dtype=jnp.float32)

    def zbody(i, _):
        zbuf_v[pl.ds(i * 16, 16)] = zeros16
        return 0
    lax.fori_loop(0, ROWS_PER_TILE // 16, zbody, 0)
    for j in range(CHUNK // 16):
        ones_v[pl.ds(j * 16, 16)] = jnp.full((16,), 1.0, dtype=jnp.float32)
    pltpu.sync_copy(zbuf_v, deg_sh.at[pl.ds(r0, ROWS_PER_TILE)])

    pltpu.sync_copy(row2d_hbm.at[pl.ds(wid * DEG_CHUNKS, DEG_CHUNKS)],
                    rowidx_v)
    plsc.subcore_barrier()

    K = 8

    def body(g, _):
        descs = [
            pltpu.async_copy(ones_v, deg_sh.at[rowidx_v.at[g * K + k]],
                             dsem, add=True)
            for k in range(K)
        ]
        for d in descs:
            d.wait()
        return 0
    lax.fori_loop(0, DEG_CHUNKS // K, body, 0)

    plsc.subcore_barrier()
    pltpu.sync_copy(deg_sh.at[pl.ds(r0, ROWS_PER_TILE)],
                    out_hbm.at[c, pl.ds(r0, ROWS_PER_TILE)])


def _sc_degree(row2d):
    mesh = plsc.VectorSubcoreMesh(core_axis_name="c", subcore_axis_name="s")
    return pl.kernel(
        _deg_body,
        out_type=jax.ShapeDtypeStruct((NUM_CORES, N_PAD), jnp.float32),
        mesh=mesh,
        scratch_types=[
            pltpu.VMEM_SHARED((N_PAD,), jnp.float32),
            pltpu.VMEM((ROWS_PER_TILE,), jnp.float32),
            pltpu.VMEM((DEG_CHUNKS, CHUNK), jnp.int32),
            pltpu.VMEM((CHUNK,), jnp.float32),
            pltpu.SemaphoreType.DMA,
        ],
        compiler_params=pltpu.CompilerParams(needs_layout_passes=False,
                                             use_tc_tiling_on_sc=False),
    )(row2d)


BLK = 512


def _dis_col(degp_blk):
    deg = jnp.sum(jnp.transpose(degp_blk), axis=1, keepdims=True)
    return jnp.where(deg > 0.0, lax.rsqrt(deg), 0.0)


def _prep_body(degp_ref, x_ref, xs_ref):
    xs = _dis_col(degp_ref[...]) * x_ref[...]
    xs_ref[0] = xs[:, :D_HALF]
    xs_ref[1] = xs[:, D_HALF:]


def _tc_prep(degp, x_pad):
    grid = (N_PAD // BLK,)
    return pl.pallas_call(
        _prep_body,
        grid=grid,
        in_specs=[
            pl.BlockSpec((NUM_CORES, BLK), lambda i: (0, i)),
            pl.BlockSpec((BLK, D_FEAT), lambda i: (i, 0)),
        ],
        out_specs=pl.BlockSpec((NUM_CORES, BLK, D_HALF), lambda i: (0, i, 0)),
        out_shape=jax.ShapeDtypeStruct((NUM_CORES, N_PAD, D_HALF),
                                       jnp.float32),
    )(degp, x_pad)


def _scat_body(xs3_hbm, row2d_hbm, col2d_hbm, zeros_hbm, out_hbm,
               acc_sh, colidx_v, rowidx_v, gbufs, gsems, ssems):
    c = lax.axis_index("c")
    s = lax.axis_index("s")
    r0 = s * ROWS_PER_TILE

    pltpu.sync_copy(zeros_hbm.at[pl.ds(r0, ROWS_PER_TILE)],
                    acc_sh.at[pl.ds(r0, ROWS_PER_TILE)])
    pltpu.sync_copy(col2d_hbm.at[pl.ds(s * SC_CHUNKS, SC_CHUNKS)], colidx_v)
    pltpu.sync_copy(row2d_hbm.at[pl.ds(s * SC_CHUNKS, SC_CHUNKS)], rowidx_v)
    plsc.subcore_barrier()

    xs_hbm = xs3_hbm.at[c]

    def gather(i):
        b = lax.rem(i, NBUF)
        pltpu.async_copy(xs_hbm.at[colidx_v.at[i]], gbufs.at[b], gsems.at[b])

    def gather_wait(i):
        b = lax.rem(i, NBUF)
        pltpu.make_async_copy(xs_hbm.at[colidx_v.at[i]], gbufs.at[b],
                              gsems.at[b]).wait()

    def scat(i):
        b = lax.rem(i, NBUF)
        pltpu.async_copy(gbufs.at[b], acc_sh.at[rowidx_v.at[i]],
                         ssems.at[b], add=True)

    def scat_wait(i):
        b = lax.rem(i, NBUF)
        pltpu.make_async_copy(gbufs.at[b], acc_sh.at[rowidx_v.at[i]],
                              ssems.at[b]).wait()

    for i in range(LOOKAHEAD):
        gather(i)

    def turn(i, _):
        @pl.when(i >= NBUF - LOOKAHEAD)
        def _():
            scat_wait(i - (NBUF - LOOKAHEAD))

        @pl.when(i + LOOKAHEAD < SC_CHUNKS)
        def _():
            gather(i + LOOKAHEAD)

        gather_wait(i)
        scat(i)
        return 0
    lax.fori_loop(0, SC_CHUNKS, turn, 0)

    for k in range(NBUF - LOOKAHEAD):
        scat_wait(SC_CHUNKS - (NBUF - LOOKAHEAD) + k)

    plsc.subcore_barrier()
    pltpu.sync_copy(acc_sh.at[pl.ds(r0, ROWS_PER_TILE)],
                    out_hbm.at[c, pl.ds(r0, ROWS_PER_TILE)])


def _sc_scatter(xs3, row2d, col2d, zeros):
    mesh = plsc.VectorSubcoreMesh(core_axis_name="c", subcore_axis_name="s")
    return pl.kernel(
        _scat_body,
        out_type=jax.ShapeDtypeStruct((NUM_CORES, N_PAD, D_HALF), jnp.float32),
        mesh=mesh,
        scratch_types=[
            pltpu.VMEM_SHARED((N_PAD, D_HALF), jnp.float32),
            pltpu.VMEM((SC_CHUNKS, CHUNK), jnp.int32),
            pltpu.VMEM((SC_CHUNKS, CHUNK), jnp.int32),
            pltpu.VMEM((NBUF, CHUNK, D_HALF), jnp.float32),
            pltpu.SemaphoreType.DMA((NBUF,)),
            pltpu.SemaphoreType.DMA((NBUF,)),
        ],
        compiler_params=pltpu.CompilerParams(needs_layout_passes=False,
                                             use_tc_tiling_on_sc=False),
    )(xs3, row2d, col2d, zeros)


def _final_body(degp_ref, x_ref, a0_ref, a1_ref, w0_ref, w1_ref, b_ref,
                out_ref):
    acc = jnp.concatenate([a0_ref[0], a1_ref[0]], axis=1)
    tx1 = -_dis_col(degp_ref[...]) * acc
    o = (jnp.dot(x_ref[...], w0_ref[...],
                 preferred_element_type=jnp.float32)
         + jnp.dot(tx1, w1_ref[...],
                   preferred_element_type=jnp.float32)
         + b_ref[...])
    out_ref[...] = jnp.where(o > 0.0, o, jnp.exp(jnp.minimum(o, 0.0)) - 1.0)


def _tc_final(degp, x_pad, acc, W0, W1, b2d):
    grid = (N_PAD // BLK,)
    return pl.pallas_call(
        _final_body,
        grid=grid,
        in_specs=[
            pl.BlockSpec((NUM_CORES, BLK), lambda i: (0, i)),
            pl.BlockSpec((BLK, D_FEAT), lambda i: (i, 0)),
            pl.BlockSpec((1, BLK, D_HALF), lambda i: (0, i, 0)),
            pl.BlockSpec((1, BLK, D_HALF), lambda i: (1, i, 0)),
            pl.BlockSpec((D_FEAT, D_FEAT), lambda i: (0, 0)),
            pl.BlockSpec((D_FEAT, D_FEAT), lambda i: (0, 0)),
            pl.BlockSpec((1, D_FEAT), lambda i: (0, 0)),
        ],
        out_specs=pl.BlockSpec((BLK, D_FEAT), lambda i: (i, 0)),
        out_shape=jax.ShapeDtypeStruct((N_PAD, D_FEAT), jnp.float32),
    )(degp, x_pad, acc, acc, W0, W1, b2d)


@jax.jit
def kernel(x, edge_index, W0, W1, b):
    row = edge_index[0]
    col = edge_index[1]
    pad_e = E_PAD - E_EDGES
    row2d = jnp.concatenate(
        [row, jnp.full((pad_e,), N_NODES, dtype=jnp.int32)]
    ).reshape(E_PAD // CHUNK, CHUNK)
    col2d = jnp.concatenate(
        [col, jnp.full((pad_e,), N_NODES, dtype=jnp.int32)]
    ).reshape(E_PAD // CHUNK, CHUNK)
    x_pad = jnp.pad(x, ((0, N_PAD - N_NODES), (0, 0)))
    zeros = jnp.zeros((N_PAD, D_HALF), dtype=jnp.float32)

    degp = _sc_degree(row2d)
    xs3 = _tc_prep(degp, x_pad)
    acc = _sc_scatter(xs3, row2d, col2d, zeros)
    out_pad = _tc_final(degp, x_pad, acc, W0, W1, b.reshape(1, D_FEAT))
    return out_pad[:N_NODES]


# speedup vs baseline: 1.3598x; 1.0060x over previous
import jax
import jax.numpy as jnp
from jax import lax
from jax.experimental import pallas as pl
from jax.experimental.pallas import tpu as pltpu
from jax.experimental.pallas import tpu_sc as plsc

N_NODES = 10000
N_PAD = 10240
E_EDGES = 320000
D_FEAT = 128
D_HALF = 64

NUM_CORES = 2
NUM_SUBCORES = 16
NUM_TILES = NUM_CORES * NUM_SUBCORES
CHUNK = 128
E_PAD = 327680
ROWS_PER_TILE = N_PAD // NUM_SUBCORES

DEG_CHUNKS = E_PAD // (NUM_TILES * CHUNK)

SC_CHUNKS = E_PAD // (NUM_SUBCORES * CHUNK)
NBUF = 5
LOOKAHEAD = 3


def _deg_body(row2d_hbm, out_hbm, deg_sh, zbuf_v, rowidx_v, ones_v, dsem):
    c = lax.axis_index("c")
    s = lax.axis_index("s")
    wid = c * NUM_SUBCORES + s
    r0 = s * ROWS_PER_TILE

    zeros16 = jnp.zeros((16,), dtype=jnp.float32)

    def zbody(i, _):
        zbuf_v[pl.ds(i * 16, 16)] = zeros16
        return 0
    lax.fori_loop(0, ROWS_PER_TILE // 16, zbody, 0)
    for j in range(CHUNK // 16):
        ones_v[pl.ds(j * 16, 16)] = jnp.full((16,), 1.0, dtype=jnp.float32)
    pltpu.sync_copy(zbuf_v, deg_sh.at[pl.ds(r0, ROWS_PER_TILE)])

    pltpu.sync_copy(row2d_hbm.at[pl.ds(wid * DEG_CHUNKS, DEG_CHUNKS)],
                    rowidx_v)
    plsc.subcore_barrier()

    K = 8

    def body(g, _):
        descs = [
            pltpu.async_copy(ones_v, deg_sh.at[rowidx_v.at[g * K + k]],
                             dsem, add=True)
            for k in range(K)
        ]
        for d in descs:
            d.wait()
        return 0
    lax.fori_loop(0, DEG_CHUNKS // K, body, 0)

    plsc.subcore_barrier()
    pltpu.sync_copy(deg_sh.at[pl.ds(r0, ROWS_PER_TILE)],
                    out_hbm.at[c, pl.ds(r0, ROWS_PER_TILE)])


def _sc_degree(row2d):
    mesh = plsc.VectorSubcoreMesh(core_axis_name="c", subcore_axis_name="s")
    return pl.kernel(
        _deg_body,
        out_type=jax.ShapeDtypeStruct((NUM_CORES, N_PAD), jnp.float32),
        mesh=mesh,
        scratch_types=[
            pltpu.VMEM_SHARED((N_PAD,), jnp.float32),
            pltpu.VMEM((ROWS_PER_TILE,), jnp.float32),
            pltpu.VMEM((DEG_CHUNKS, CHUNK), jnp.int32),
            pltpu.VMEM((CHUNK,), jnp.float32),
            pltpu.SemaphoreType.DMA,
        ],
        compiler_params=pltpu.CompilerParams(needs_layout_passes=False,
                                             use_tc_tiling_on_sc=False),
    )(row2d)


BLK = 512


def _dis_col(degp_blk):
    deg = jnp.sum(jnp.transpose(degp_blk), axis=1, keepdims=True)
    return jnp.where(deg > 0.0, lax.rsqrt(deg), 0.0)


def _prep_body(degp_ref, x_ref, xs_ref):
    xs = _dis_col(degp_ref[...]) * x_ref[...]
    xs_ref[0] = xs[:, :D_HALF]
    xs_ref[1] = xs[:, D_HALF:]


def _tc_prep(degp, x_pad):
    grid = (N_PAD // BLK,)
    return pl.pallas_call(
        _prep_body,
        grid=grid,
        in_specs=[
            pl.BlockSpec((NUM_CORES, BLK), lambda i: (0, i)),
            pl.BlockSpec((BLK, D_FEAT), lambda i: (i, 0)),
        ],
        out_specs=pl.BlockSpec((NUM_CORES, BLK, D_HALF), lambda i: (0, i, 0)),
        out_shape=jax.ShapeDtypeStruct((NUM_CORES, N_PAD, D_HALF),
                                       jnp.float32),
    )(degp, x_pad)


def _scat_body(xs3_hbm, row2d_hbm, col2d_hbm, zeros_hbm, out_hbm,
               acc_sh, colidx_v, rowidx_v, gbufs, gsems, ssems):
    c = lax.axis_index("c")
    s = lax.axis_index("s")
    r0 = s * ROWS_PER_TILE

    pltpu.sync_copy(zeros_hbm.at[pl.ds(r0, ROWS_PER_TILE)],
                    acc_sh.at[pl.ds(r0, ROWS_PER_TILE)])
    pltpu.sync_copy(col2d_hbm.at[pl.ds(s * SC_CHUNKS, SC_CHUNKS)], colidx_v)
    pltpu.sync_copy(row2d_hbm.at[pl.ds(s * SC_CHUNKS, SC_CHUNKS)], rowidx_v)
    plsc.subcore_barrier()

    xs_hbm = xs3_hbm.at[c]

    def gather(i):
        b = lax.rem(i, NBUF)
        pltpu.async_copy(xs_hbm.at[colidx_v.at[i]], gbufs.at[b], gsems.at[b])

    def gather_wait(i):
        b = lax.rem(i, NBUF)
        pltpu.make_async_copy(xs_hbm.at[colidx_v.at[i]], gbufs.at[b],
                              gsems.at[b]).wait()

    def scat(i):
        b = lax.rem(i, NBUF)
        pltpu.async_copy(gbufs.at[b], acc_sh.at[rowidx_v.at[i]],
                         ssems.at[b], add=True)

    def scat_wait(i):
        b = lax.rem(i, NBUF)
        pltpu.make_async_copy(gbufs.at[b], acc_sh.at[rowidx_v.at[i]],
                              ssems.at[b]).wait()

    for i in range(LOOKAHEAD):
        gather(i)

    def turn(i, _):
        @pl.when(i >= NBUF - LOOKAHEAD)
        def _():
            scat_wait(i - (NBUF - LOOKAHEAD))

        @pl.when(i + LOOKAHEAD < SC_CHUNKS)
        def _():
            gather(i + LOOKAHEAD)

        gather_wait(i)
        scat(i)
        return 0
    lax.fori_loop(0, SC_CHUNKS, turn, 0)

    for k in range(NBUF - LOOKAHEAD):
        scat_wait(SC_CHUNKS - (NBUF - LOOKAHEAD) + k)

    plsc.subcore_barrier()
    pltpu.sync_copy(acc_sh.at[pl.ds(r0, ROWS_PER_TILE)],
                    out_hbm.at[c, pl.ds(r0, ROWS_PER_TILE)])


def _sc_scatter(xs3, row2d, col2d, zeros):
    mesh = plsc.VectorSubcoreMesh(core_axis_name="c", subcore_axis_name="s")
    return pl.kernel(
        _scat_body,
        out_type=jax.ShapeDtypeStruct((NUM_CORES, N_PAD, D_HALF), jnp.float32),
        mesh=mesh,
        scratch_types=[
            pltpu.VMEM_SHARED((N_PAD, D_HALF), jnp.float32),
            pltpu.VMEM((SC_CHUNKS, CHUNK), jnp.int32),
            pltpu.VMEM((SC_CHUNKS, CHUNK), jnp.int32),
            pltpu.VMEM((NBUF, CHUNK, D_HALF), jnp.float32),
            pltpu.SemaphoreType.DMA((NBUF,)),
            pltpu.SemaphoreType.DMA((NBUF,)),
        ],
        compiler_params=pltpu.CompilerParams(needs_layout_passes=False,
                                             use_tc_tiling_on_sc=False),
    )(xs3, row2d, col2d, zeros)


def _final_body(degp_ref, x_ref, a0_ref, a1_ref, w0_ref, w1_ref, b_ref,
                out_ref):
    acc = jnp.concatenate([a0_ref[0], a1_ref[0]], axis=1)
    tx1 = -_dis_col(degp_ref[...]) * acc
    o = (jnp.dot(x_ref[...], w0_ref[...],
                 preferred_element_type=jnp.float32)
         + jnp.dot(tx1, w1_ref[...],
                   preferred_element_type=jnp.float32)
         + b_ref[...])
    out_ref[...] = jnp.where(o > 0.0, o, jnp.exp(jnp.minimum(o, 0.0)) - 1.0)


def _tc_final(degp, x_pad, acc, W0, W1, b2d):
    grid = (N_PAD // BLK,)
    return pl.pallas_call(
        _final_body,
        grid=grid,
        in_specs=[
            pl.BlockSpec((NUM_CORES, BLK), lambda i: (0, i)),
            pl.BlockSpec((BLK, D_FEAT), lambda i: (i, 0)),
            pl.BlockSpec((1, BLK, D_HALF), lambda i: (0, i, 0)),
            pl.BlockSpec((1, BLK, D_HALF), lambda i: (1, i, 0)),
            pl.BlockSpec((D_FEAT, D_FEAT), lambda i: (0, 0)),
            pl.BlockSpec((D_FEAT, D_FEAT), lambda i: (0, 0)),
            pl.BlockSpec((1, D_FEAT), lambda i: (0, 0)),
        ],
        out_specs=pl.BlockSpec((BLK, D_FEAT), lambda i: (i, 0)),
        out_shape=jax.ShapeDtypeStruct((N_PAD, D_FEAT), jnp.float32),
    )(degp, x_pad, acc, acc, W0, W1, b2d)


@jax.jit
def kernel(x, edge_index, W0, W1, b):
    row = edge_index[0]
    col = edge_index[1]
    pad_e = E_PAD - E_EDGES
    row2d = jnp.concatenate(
        [row, jnp.full((pad_e,), N_NODES, dtype=jnp.int32)]
    ).reshape(E_PAD // CHUNK, CHUNK)
    col2d = jnp.concatenate(
        [col, jnp.full((pad_e,), N_NODES, dtype=jnp.int32)]
    ).reshape(E_PAD // CHUNK, CHUNK)
    x_pad = jnp.pad(x, ((0, N_PAD - N_NODES), (0, 0)))
    zeros = jnp.zeros((N_PAD, D_HALF), dtype=jnp.float32)

    degp = _sc_degree(row2d)
    xs3 = _tc_prep(degp, x_pad)
    acc = _sc_scatter(xs3, row2d, col2d, zeros)
    out_pad = _tc_final(degp, x_pad, acc, W0, W1, b.reshape(1, D_FEAT))
    return out_pad[:N_NODES]
